# baseline (speedup 1.0000x reference)
import jax
import jax.numpy as jnp
from jax import lax
from jax.experimental import pallas as pl
from jax.experimental.pallas import tpu as pltpu
from jax.experimental.pallas import tpu_sc as plsc

_PQ, _PH, _PD = 192, 384, 768
_B, _T = 256, 196
_NROWS = _B * _T
_NC, _NS = 2, 16
_NW = _NC * _NS
_RPW = _NROWS // _NW
_R = 56
_NCHUNK = _RPW // _R
_NV = _PD // 16


def _src_vec(v):
  p = lax.iota(jnp.int32, 16) + jnp.int32(16 * v)
  if v < _PQ // 16 or v >= (_PH + _PQ) // 16:
    return p
  m = p - _PQ
  return _PQ + (m >> 1) + (m & 1) * _PQ


def _body(x_hbm, o_hbm, in_v, out_v):
  wid = lax.axis_index("s") * _NC + lax.axis_index("c")
  base = wid * _RPW

  def chunk(c, carry):
    off0 = (base + c * _R) * _PD
    pltpu.sync_copy(x_hbm.at[pl.ds(off0, _R * _PD)], in_v)
    for v in range(_NV):
      src = _src_vec(v)

      @plsc.parallel_loop(0, _R, 1, unroll=8)
      def _row(r):
        out_v[pl.ds(r * _PD + 16 * v, 16)] = plsc.load_gather(
            in_v, [src + r * _PD])

    pltpu.sync_copy(out_v, o_hbm.at[pl.ds(off0, _R * _PD)])
    return carry

  lax.fori_loop(0, _NCHUNK, chunk, 0)


def kernel(inputs):
  x = inputs.reshape(_NROWS * _PD)
  mesh = plsc.VectorSubcoreMesh(
      core_axis_name="c", subcore_axis_name="s",
      num_cores=_NC, num_subcores=_NS)
  out = pl.kernel(
      _body,
      out_type=jax.ShapeDtypeStruct((_NROWS * _PD,), jnp.float32),
      mesh=mesh,
      compiler_params=pltpu.CompilerParams(needs_layout_passes=False),
      scratch_types=[
          pltpu.VMEM((_R * _PD,), jnp.float32),
          pltpu.VMEM((_R * _PD,), jnp.float32),
      ],
  )(x)
  return out.reshape(_B, _T, _PD)


# speedup vs baseline: 1.1098x; 1.1098x over previous
import jax
import jax.numpy as jnp
from jax import lax
from jax.experimental import pallas as pl
from jax.experimental.pallas import tpu as pltpu
from jax.experimental.pallas import tpu_sc as plsc

_PQ, _PH, _PD = 192, 384, 768
_B, _T = 256, 196
_NROWS = _B * _T
_NC, _NS = 2, 16
_NW = _NC * _NS
_RPW = _NROWS // _NW
_R = 28
_CH = _RPW // _R
_CW = _R * _PD
_NV = _PD // 16


def _src_vec(v):
  p = lax.iota(jnp.int32, 16) + jnp.int32(16 * v)
  m = p - _PQ
  return _PQ + (m >> 1) + (m & 1) * _PQ


def _compute_chunk(in_v, out_v):
  for v in range(_NV):
    lo = 16 * v
    if _PQ // 16 <= v < (_PH + _PQ) // 16:
      src = _src_vec(v)

      @plsc.parallel_loop(0, _R, 1, unroll=7)
      def _row(r):
        out_v[pl.ds(r * _PD + lo, 16)] = plsc.load_gather(
            in_v, [src + r * _PD])
    else:

      @plsc.parallel_loop(0, _R, 1, unroll=7)
      def _row(r):
        out_v[pl.ds(r * _PD + lo, 16)] = in_v[pl.ds(r * _PD + lo, 16)]


def _body(x_hbm, o_hbm, in0, in1, out0, out1, si0, si1, so0, so1):
  wid = lax.axis_index("s") * _NC + lax.axis_index("c")
  base = wid * (_RPW * _PD)
  ins, outs = (in0, in1), (out0, out1)
  sis, sos = (si0, si1), (so0, so1)

  def _off(c):
    return base + c * _CW

  def _wait_read(b):
    pltpu.make_async_copy(x_hbm.at[pl.ds(0, _CW)], ins[b], sis[b]).wait()

  def _wait_write(b):
    pltpu.make_async_copy(outs[b], o_hbm.at[pl.ds(0, _CW)], sos[b]).wait()

  pltpu.async_copy(x_hbm.at[pl.ds(_off(0), _CW)], in0, si0)
  pltpu.async_copy(x_hbm.at[pl.ds(_off(1), _CW)], in1, si1)

  def pair(g, carry):
    for b in (0, 1):
      c = 2 * g + b
      _wait_read(b)

      @pl.when(c >= 2)
      def _():
        _wait_write(b)

      _compute_chunk(ins[b], outs[b])
      pltpu.async_copy(outs[b], o_hbm.at[pl.ds(_off(c), _CW)], sos[b])

      @pl.when(c + 2 < _CH)
      def _():
        pltpu.async_copy(x_hbm.at[pl.ds(_off(c + 2), _CW)], ins[b], sis[b])

    return carry

  lax.fori_loop(0, _CH // 2, pair, 0)

  _wait_write(0)
  _wait_write(1)


def kernel(inputs):
  x = inputs.reshape(_NROWS * _PD)
  mesh = plsc.VectorSubcoreMesh(
      core_axis_name="c", subcore_axis_name="s",
      num_cores=_NC, num_subcores=_NS)
  out = pl.kernel(
      _body,
      out_type=jax.ShapeDtypeStruct((_NROWS * _PD,), jnp.float32),
      mesh=mesh,
      compiler_params=pltpu.CompilerParams(needs_layout_passes=False),
      scratch_types=[
          pltpu.VMEM((_CW,), jnp.float32),
          pltpu.VMEM((_CW,), jnp.float32),
          pltpu.VMEM((_CW,), jnp.float32),
          pltpu.VMEM((_CW,), jnp.float32),
          pltpu.SemaphoreType.DMA,
          pltpu.SemaphoreType.DMA,
          pltpu.SemaphoreType.DMA,
          pltpu.SemaphoreType.DMA,
      ],
  )(x)
  return out.reshape(_B, _T, _PD)


# speedup vs baseline: 5.7019x; 5.1377x over previous
import jax
import jax.numpy as jnp
from jax import lax
from jax.experimental import pallas as pl
from jax.experimental.pallas import tpu as pltpu
from jax.experimental.pallas import tpu_sc as plsc

_PQ, _PH, _PD = 192, 384, 768
_B, _T = 256, 196
_NROWS = _B * _T
_NC, _NS = 2, 16
_NW = _NC * _NS
_RPW = _NROWS // _NW
_R = 32
_CH = _RPW // _R
_NV = _PD // 16


def _src_vec(v):
  p = lax.iota(jnp.int32, 16) + jnp.int32(16 * v)
  m = p - _PQ
  return _PQ + (m >> 1) + (m & 1) * _PQ


def _compute_chunk(in_v, out_v):
  @plsc.parallel_loop(0, _NV, 1)
  def _v(v):
    p = lax.iota(jnp.int32, 16) + 16 * v
    m = p - _PQ
    src = jnp.where((p >= _PQ) & (p < _PH + _PQ),
                    _PQ + (m >> 1) + (m & 1) * _PQ, p)
    lo = 16 * v

    @plsc.parallel_loop(0, _R, 1, unroll=8)
    def _row(r):
      rv = jnp.full((16,), r, jnp.int32)
      out_v[r, pl.ds(lo, 16)] = plsc.load_gather(in_v, [rv, src])


def _body(x_hbm, o_hbm, in0, in1, out0, out1, si0, si1, so0, so1):
  wid = lax.axis_index("s") * _NC + lax.axis_index("c")
  base = wid * _RPW
  ins, outs = (in0, in1), (out0, out1)
  sis, sos = (si0, si1), (so0, so1)

  def _row0(c):
    return base + c * _R

  def _wait_read(b):
    pltpu.make_async_copy(x_hbm.at[pl.ds(0, _R)], ins[b], sis[b]).wait()

  def _wait_write(b):
    pltpu.make_async_copy(outs[b], o_hbm.at[pl.ds(0, _R)], sos[b]).wait()

  pltpu.async_copy(x_hbm.at[pl.ds(_row0(0), _R)], in0, si0)
  pltpu.async_copy(x_hbm.at[pl.ds(_row0(1), _R)], in1, si1)

  def pair(g, carry):
    for b in (0, 1):
      c = 2 * g + b
      _wait_read(b)

      @pl.when(c >= 2)
      def _():
        _wait_write(b)

      _compute_chunk(ins[b], outs[b])
      pltpu.async_copy(outs[b], o_hbm.at[pl.ds(_row0(c), _R)], sos[b])

      @pl.when(c + 2 < _CH)
      def _():
        pltpu.async_copy(x_hbm.at[pl.ds(_row0(c + 2), _R)], ins[b], sis[b])

    return carry

  lax.fori_loop(0, _CH // 2, pair, 0)

  c_last = _CH - 1
  _wait_read(0)
  _wait_write(0)
  _compute_chunk(in0, out0)
  pltpu.async_copy(out0, o_hbm.at[pl.ds(_row0(c_last), _R)], so0)

  _wait_write(0)
  _wait_write(1)


def kernel(inputs):
  x = inputs.transpose(1, 0, 2).reshape(_NROWS, _PD)
  mesh = plsc.VectorSubcoreMesh(
      core_axis_name="c", subcore_axis_name="s",
      num_cores=_NC, num_subcores=_NS)
  out = pl.kernel(
      _body,
      out_type=jax.ShapeDtypeStruct((_NROWS, _PD), jnp.float32),
      mesh=mesh,
      compiler_params=pltpu.CompilerParams(needs_layout_passes=False),
      scratch_types=[
          pltpu.VMEM((_R, _PD), jnp.float32),
          pltpu.VMEM((_R, _PD), jnp.float32),
          pltpu.VMEM((_R, _PD), jnp.float32),
          pltpu.VMEM((_R, _PD), jnp.float32),
          pltpu.SemaphoreType.DMA,
          pltpu.SemaphoreType.DMA,
          pltpu.SemaphoreType.DMA,
          pltpu.SemaphoreType.DMA,
      ],
  )(x)
  return out.reshape(_T, _B, _PD).transpose(1, 0, 2)


# speedup vs baseline: 5.8603x; 1.0278x over previous
import jax
import jax.numpy as jnp
from jax import lax
from jax.experimental import pallas as pl
from jax.experimental.pallas import tpu as pltpu
from jax.experimental.pallas import tpu_sc as plsc

_PQ, _PH, _PD = 192, 384, 768
_B, _T = 256, 196
_NROWS = _B * _T
_NC, _NS = 2, 16
_NW = _NC * _NS
_RPW = _NROWS // _NW
_R = 32
_CH = _RPW // _R
_NV = _PD // 16
_NIN, _NOUT = 3, 2


def _compute_chunk(in_v, out_v):
  @plsc.parallel_loop(0, _NV, 1)
  def _v(v):
    p = lax.iota(jnp.int32, 16) + 16 * v
    m = p - _PQ
    src = jnp.where((p >= _PQ) & (p < _PH + _PQ),
                    _PQ + (m >> 1) + (m & 1) * _PQ, p)
    lo = 16 * v

    @plsc.parallel_loop(0, _R, 1, unroll=8)
    def _row(r):
      rv = jnp.full((16,), r, jnp.int32)
      out_v[r, pl.ds(lo, 16)] = plsc.load_gather(in_v, [rv, src])


def _body(x_hbm, o_hbm, in0, in1, in2, out0, out1, si0, si1, si2, so0, so1):
  wid = lax.axis_index("s") * _NC + lax.axis_index("c")
  base = wid * _RPW
  ins, outs = (in0, in1, in2), (out0, out1)
  sis, sos = (si0, si1, si2), (so0, so1)

  def _row0(c):
    return base + c * _R

  def _wait_read(i):
    pltpu.make_async_copy(x_hbm.at[pl.ds(0, _R)], ins[i], sis[i]).wait()

  def _wait_write(i):
    pltpu.make_async_copy(outs[i], o_hbm.at[pl.ds(0, _R)], sos[i]).wait()

  def _start_read(c, i):
    pltpu.async_copy(x_hbm.at[pl.ds(_row0(c), _R)], ins[i], sis[i])

  def _start_write(c, i):
    pltpu.async_copy(outs[i], o_hbm.at[pl.ds(_row0(c), _R)], sos[i])

  for c in range(_NIN):
    _start_read(c, c)

  def group(g, carry):
    for b in range(6):
      c = 6 * g + b
      bi, bo = b % _NIN, b % _NOUT
      _wait_read(bi)

      @pl.when(c >= _NOUT)
      def _():
        _wait_write(bo)

      _compute_chunk(ins[bi], outs[bo])

      @pl.when(c + _NIN < _CH)
      def _():
        _start_read(c + _NIN, bi)

      _start_write(c, bo)

    return carry

  lax.fori_loop(0, _CH // 6, group, 0)

  c_last = _CH - 1
  _wait_read(0)
  _wait_write(0)
  _compute_chunk(in0, out0)
  _start_write(c_last, 0)

  _wait_write(0)
  _wait_write(1)


def kernel(inputs):
  x = inputs.transpose(1, 0, 2).reshape(_NROWS, _PD)
  mesh = plsc.VectorSubcoreMesh(
      core_axis_name="c", subcore_axis_name="s",
      num_cores=_NC, num_subcores=_NS)
  out = pl.kernel(
      _body,
      out_type=jax.ShapeDtypeStruct((_NROWS, _PD), jnp.float32),
      mesh=mesh,
      compiler_params=pltpu.CompilerParams(needs_layout_passes=False),
      scratch_types=[
          pltpu.VMEM((_R, _PD), jnp.float32),
          pltpu.VMEM((_R, _PD), jnp.float32),
          pltpu.VMEM((_R, _PD), jnp.float32),
          pltpu.VMEM((_R, _PD), jnp.float32),
          pltpu.VMEM((_R, _PD), jnp.float32),
          pltpu.SemaphoreType.DMA,
          pltpu.SemaphoreType.DMA,
          pltpu.SemaphoreType.DMA,
          pltpu.SemaphoreType.DMA,
          pltpu.SemaphoreType.DMA,
      ],
  )(x)
  return out.reshape(_T, _B, _PD).transpose(1, 0, 2)
